# baseline (speedup 1.0000x reference)
import functools

import jax
import jax.numpy as jnp
from jax import lax
from jax.experimental import pallas as pl
from jax.experimental.pallas import tpu as pltpu
from jax.experimental.pallas import tpu_sc as plsc

VOCAB = 1000000
D = 32
C = 20
NEG = 5
B = 16384

NC = 2
NS = 16
NW = NC * NS
BPW = B // NW
CB = 16
NCHUNK = BPW // CB
IPB = C + 1 + NEG
ROWS = CB * IPB
RPAD = 512
IBLK = 8


def _sc_body(idx_hbm, emb_hbm, out_hbm, comb_v, rows_v, acc_v, sem0, sem1):
    wid = lax.axis_index("s") * NC + lax.axis_index("c")
    sems = (sem0, sem1)
    iota = lax.iota(jnp.int32, 16)

    def fire(g, slot):
        pltpu.sync_copy(idx_hbm.at[wid, pl.ds(g * IBLK, IBLK)],
                        comb_v.at[slot])
        handles = []
        for j in range(3):
            handles.append(pltpu.async_copy(
                emb_hbm.at[comb_v.at[slot, j]],
                rows_v.at[slot, pl.ds(j * 128, 128)],
                sems[slot]))
        handles.append(pltpu.async_copy(
            emb_hbm.at[comb_v.at[slot, 3, pl.ds(0, 32)]],
            rows_v.at[slot, pl.ds(384, 32)],
            sems[slot]))
        return handles

    def compute(slot, accs):
        rows = rows_v.at[slot]
        comb = comb_v.at[slot]

        def pick(q, half):
            off = plsc.load_gather(
                comb, [jnp.full((16,), 4 + (q >> 7), jnp.int32),
                       jnp.full((16,), q & 127, jnp.int32)])
            return plsc.load_gather(
                rows, [jnp.full((16,), q, jnp.int32), off + (iota + half)])

        def body(b, accs):
            ap, a0, a1, a2, a3, a4 = accs
            cb = b * C
            ulo = pick(cb, 0)
            uhi = pick(cb, 16)
            for c in range(1, C):
                ulo = ulo + pick(cb + c, 0)
                uhi = uhi + pick(cb + c, 16)

            def dot(r):
                return ulo * pick(r, 0) + uhi * pick(r, 16)

            ap = ap + dot(CB * C + b)
            nb = CB * (C + 1) + b * NEG
            a0 = a0 + dot(nb)
            a1 = a1 + dot(nb + 1)
            a2 = a2 + dot(nb + 2)
            a3 = a3 + dot(nb + 3)
            a4 = a4 + dot(nb + 4)
            return (ap, a0, a1, a2, a3, a4)

        return lax.fori_loop(0, CB, body, accs)

    def drain(slot):
        for j in range(3):
            pltpu.make_async_copy(
                emb_hbm.at[comb_v.at[slot, j]],
                rows_v.at[slot, pl.ds(j * 128, 128)],
                sems[slot]).wait()
        pltpu.make_async_copy(
            emb_hbm.at[comb_v.at[slot, 3, pl.ds(0, 32)]],
            rows_v.at[slot, pl.ds(384, 32)],
            sems[slot]).wait()

    zero = jnp.zeros((16,), jnp.float32)
    accs = (zero, zero, zero, zero, zero, zero)
    fire(0, 0)

    def outer(g2, accs):
        base = g2 * 2
        fire(base + 1, 1)
        drain(0)
        accs = compute(0, accs)

        @pl.when(base + 2 < NCHUNK)
        def _():
            fire(base + 2, 0)

        drain(1)
        return compute(1, accs)

    accs = lax.fori_loop(0, NCHUNK // 2, outer, accs)
    for i in range(6):
        acc_v[i, 0:16] = accs[i]
    pltpu.sync_copy(acc_v, out_hbm.at[wid])


@functools.cache
def _cbow_sc():
    return pl.kernel(
        _sc_body,
        out_type=jax.ShapeDtypeStruct((NW, 8, 128), jnp.float32),
        mesh=plsc.VectorSubcoreMesh(core_axis_name="c", subcore_axis_name="s",
                                    num_cores=NC, num_subcores=NS),
        scratch_types=[
            pltpu.VMEM((2, IBLK, 128), jnp.int32),
            pltpu.VMEM((2, ROWS, 128), jnp.float32),
            pltpu.VMEM((8, 128), jnp.float32),
            pltpu.SemaphoreType.DMA,
            pltpu.SemaphoreType.DMA,
        ],
        compiler_params=pltpu.CompilerParams(use_tc_tiling_on_sc=True,
                                             needs_layout_passes=False),
    )


def kernel(pos_u, pos_w, neg_w, emb):
    pos_u = pos_u.astype(jnp.int32)
    pos_w = pos_w.astype(jnp.int32)
    neg_w = neg_w.astype(jnp.int32)
    ctx = pos_u.reshape(NW, NCHUNK, CB * C)
    pw = pos_w.reshape(NW, NCHUNK, CB)
    ng = neg_w.reshape(NW, NCHUNK, CB * NEG)
    pad = jnp.zeros((NW, NCHUNK, RPAD - ROWS), jnp.int32)
    flat = jnp.concatenate([ctx, pw, ng, pad], axis=-1)
    packed = flat >> 2
    offs = (flat & 3) << 5
    blk = jnp.concatenate(
        [packed.reshape(NW, NCHUNK, 4, 128), offs.reshape(NW, NCHUNK, 4, 128)],
        axis=2)
    allidx = blk.reshape(NW, NCHUNK * IBLK, 128)
    emb4 = emb.reshape(VOCAB // 4, 4 * D)
    partials = _cbow_sc()(allidx, emb4)
    s = jnp.sum(partials[:, 0:6, 0:16], axis=(0, 2))
    return -jax.nn.log_sigmoid(s[0]) - jnp.sum(jax.nn.log_sigmoid(-s[1:]))


# speedup vs baseline: 2.5693x; 2.5693x over previous
import functools

import jax
import jax.numpy as jnp
from jax import lax
from jax.experimental import pallas as pl
from jax.experimental.pallas import tpu as pltpu
from jax.experimental.pallas import tpu_sc as plsc

VOCAB = 1000000
D = 32
C = 20
NEG = 5
B = 16384

PBLK = 1536
PGRID = 163
VP = PGRID * PBLK

NC = 2
NS = 16
NW = NC * NS
BPW = B // NW
CB = 64
NCHUNK = BPW // CB
IPB = C + 1 + NEG
ROWS = CB * IPB
NSTREAM = ROWS // 128
ISTRIDE = 16


def _pack_body(t0, t1, t2, t3, o_ref):
    stack = jnp.concatenate([t0[...], t1[...], t2[...], t3[...]], axis=0)
    o_ref[...] = stack.T


@functools.cache
def _pack_tc():
    return pl.pallas_call(
        _pack_body,
        grid=(PGRID,),
        in_specs=[pl.BlockSpec((D, PBLK), lambda i, q=q: (0, q * PGRID + i))
                  for q in range(4)],
        out_specs=pl.BlockSpec((PBLK, 128), lambda i: (i, 0)),
        out_shape=jax.ShapeDtypeStruct((VP, 128), jnp.float32),
    )


def _sc_body(idx_hbm, emb_hbm, out_hbm, idx_v, rows_v, acc_v, sem0, sem1):
    wid = lax.axis_index("s") * NC + lax.axis_index("c")
    sems = (sem0, sem1)

    def fire(g, slot):
        pltpu.sync_copy(idx_hbm.at[wid, pl.ds(g * ISTRIDE, ISTRIDE)],
                        idx_v.at[slot])
        handles = []
        for j in range(NSTREAM):
            handles.append(pltpu.async_copy(
                emb_hbm.at[idx_v.at[slot, j]],
                rows_v.at[slot, pl.ds(j * 128, 128)],
                sems[slot]))
        return handles

    def compute(slot, accs):
        def body(b, accs):
            ap, a0, a1, a2, a3, a4 = accs
            cb = b * C
            ulo = rows_v[slot, cb, 0:16]
            uhi = rows_v[slot, cb, 16:32]
            for c in range(1, C):
                ulo = ulo + rows_v[slot, cb + c, 0:16]
                uhi = uhi + rows_v[slot, cb + c, 16:32]

            def dot(r):
                return ulo * rows_v[slot, r, 0:16] + uhi * rows_v[slot, r, 16:32]

            ap = ap + dot(CB * C + b)
            nb = CB * (C + 1) + b * NEG
            a0 = a0 + dot(nb)
            a1 = a1 + dot(nb + 1)
            a2 = a2 + dot(nb + 2)
            a3 = a3 + dot(nb + 3)
            a4 = a4 + dot(nb + 4)
            return (ap, a0, a1, a2, a3, a4)

        return lax.fori_loop(0, CB, body, accs)

    zero = jnp.zeros((16,), jnp.float32)
    accs = (zero, zero, zero, zero, zero, zero)
    handles = fire(0, 0)
    for g in range(NCHUNK):
        slot = g & 1
        nxt = fire(g + 1, slot ^ 1) if g + 1 < NCHUNK else None
        for h in handles:
            h.wait()
        accs = compute(slot, accs)
        handles = nxt
    for i in range(6):
        acc_v[i, 0:16] = accs[i]
    pltpu.sync_copy(acc_v, out_hbm.at[wid])


@functools.cache
def _cbow_sc():
    return pl.kernel(
        _sc_body,
        out_type=jax.ShapeDtypeStruct((NW, 8, 128), jnp.float32),
        mesh=plsc.VectorSubcoreMesh(core_axis_name="c", subcore_axis_name="s",
                                    num_cores=NC, num_subcores=NS),
        scratch_types=[
            pltpu.VMEM((2, ISTRIDE, 128), jnp.int32),
            pltpu.VMEM((2, ROWS, D), jnp.float32),
            pltpu.VMEM((8, 128), jnp.float32),
            pltpu.SemaphoreType.DMA,
            pltpu.SemaphoreType.DMA,
        ],
        compiler_params=pltpu.CompilerParams(use_tc_tiling_on_sc=False),
    )


def kernel(pos_u, pos_w, neg_w, emb):
    pos_u = pos_u.astype(jnp.int32)
    pos_w = pos_w.astype(jnp.int32)
    neg_w = neg_w.astype(jnp.int32)
    ctx = pos_u.reshape(NW, NCHUNK, CB * C)
    pw = pos_w.reshape(NW, NCHUNK, CB)
    ng = neg_w.reshape(NW, NCHUNK, CB * NEG)
    pad = jnp.zeros((NW, NCHUNK, (ISTRIDE - NSTREAM) * 128), jnp.int32)
    flat = jnp.concatenate([ctx, pw, ng, pad], axis=-1)
    q = flat // VP
    lin = ((flat - q * VP) << 2) + q
    allidx = lin.reshape(NW, NCHUNK * ISTRIDE, 128)
    tview = emb.T
    emb4 = _pack_tc()(tview, tview, tview, tview)
    emb_lin = emb4.reshape(4 * VP, D)
    partials = _cbow_sc()(allidx, emb_lin)
    s = jnp.sum(partials[:, 0:6, 0:16], axis=(0, 2))
    return -jax.nn.log_sigmoid(s[0]) - jnp.sum(jax.nn.log_sigmoid(-s[1:]))
